# baseline (speedup 1.0000x reference)
import functools

import jax
import jax.numpy as jnp
from jax import lax
from jax.experimental import pallas as pl
from jax.experimental.pallas import tpu as pltpu
from jax.experimental.pallas import tpu_sc as plsc

_INFO = plsc.get_sparse_core_info()
_NC = _INFO.num_cores
_NS = _INFO.num_subcores

_V = 1000000
_D = 32
_R = 50
_N = 16384
_NBLK = _V // 128
_TAIL = _V - _NBLK * 128
_SROWS = _V * _D // 128
_NW = _NC * _NS
_NPW = _N // _NW
_C = 256


def _make_kernel():
    mesh = plsc.VectorSubcoreMesh(core_axis_name="c", subcore_axis_name="s")

    @functools.partial(
        pl.kernel,
        mesh=mesh,
        out_type=jax.ShapeDtypeStruct((_R, _D, _N), jnp.float32),
        scratch_types=[
            pltpu.HBM((_NC, _SROWS, 128), jnp.float32),
            pltpu.VMEM((_D, 128), jnp.float32),
            pltpu.VMEM((_D, 128), jnp.float32),
            pltpu.VMEM((_C,), jnp.int32),
            pltpu.VMEM((_C,), jnp.int32),
            pltpu.VMEM((_C,), jnp.int32),
            pltpu.VMEM((_C, 128), jnp.float32),
            pltpu.VMEM((_D, _C), jnp.float32),
            pltpu.SemaphoreType.DMA,
        ],
        compiler_params=pltpu.CompilerParams(needs_layout_passes=False),
    )
    def k(wt_t, idx_t, tail_rows, out_t, scr, tbuf, obuf1, ibuf, qbuf, cbuf,
          gbuf, obuf2, sem_g):
        c = lax.axis_index("c")
        s = lax.axis_index("s")
        row_lo = lax.iota(jnp.int32, 16)
        row_hi = row_lo + 16

        def shuffle_block(n_rows):
            for k_ in range(n_rows):
                for lg in range(8):
                    l0 = lg * 16
                    rv = row_lo if (l0 % 32) == 0 else row_hi
                    cv = jnp.full((16,), 4 * k_ + l0 // 32, jnp.int32)
                    obuf1[k_, pl.ds(l0, 16)] = plsc.load_gather(tbuf, [rv, cv])

        def p1_body(t, carry):
            g = s + 16 * t

            @pl.when(g < _NBLK)
            def _():
                pltpu.sync_copy(wt_t.at[:, pl.ds(g * 128, 128)], tbuf)
                shuffle_block(32)
                pltpu.sync_copy(obuf1, scr.at[c, pl.ds(32 * g, 32), :])

            return carry

        lax.fori_loop(0, (_NBLK + _NS - 1) // _NS, p1_body, 0)

        @pl.when(s == 0)
        def _tail():
            pltpu.sync_copy(tail_rows, obuf1.at[pl.ds(0, _TAIL * _D // 128), :])
            pltpu.sync_copy(obuf1.at[pl.ds(0, _TAIL * _D // 128), :],
                            scr.at[c, pl.ds(32 * _NBLK, _TAIL * _D // 128), :])

        plsc.subcore_barrier()

        wid = s * _NC + c
        n0w = wid * _NPW
        scr_c = scr.at[c]

        def p2_body(i, carry):
            r = i // (_NPW // _C)
            ch = i % (_NPW // _C)
            n0 = n0w + ch * _C
            pltpu.sync_copy(idx_t.at[r, pl.ds(n0, _C)], ibuf)
            for m in range(_C // 16):
                v = ibuf[pl.ds(m * 16, 16)]
                qbuf[pl.ds(m * 16, 16)] = lax.shift_right_logical(v, 2)
                cbuf[pl.ds(m * 16, 16)] = lax.shift_left(
                    jnp.bitwise_and(v, 3), 5)
            for h in range(_C // 128):
                pltpu.async_copy(
                    scr_c.at[qbuf.at[pl.ds(h * 128, 128)]],
                    gbuf.at[pl.ds(h * 128, 128), :], sem_g)
            for h in range(_C // 128):
                pltpu.make_async_copy(
                    scr_c.at[pl.ds(0, 128), :],
                    gbuf.at[pl.ds(h * 128, 128), :], sem_g).wait()

            def mg_body(mg, carry2):
                m0 = mg * 16
                rv = row_lo + m0
                cb = cbuf[pl.ds(m0, 16)]
                for j in range(_D):
                    obuf2[j, pl.ds(m0, 16)] = plsc.load_gather(
                        gbuf, [rv, cb + j])
                return carry2

            lax.fori_loop(0, _C // 16, mg_body, 0)
            pltpu.sync_copy(obuf2, out_t.at[r, :, pl.ds(n0, _C)])
            return carry

        lax.fori_loop(0, _R * (_NPW // _C), p2_body, 0)

    return k


_GATHER = _make_kernel()


def kernel(weights, indices):
    idx_t = indices.astype(jnp.int32).T
    tail_rows = weights[_NBLK * 128:].reshape(_TAIL * _D // 128, 128)
    out_t = _GATHER(weights.T, idx_t, tail_rows)
    return out_t.transpose(2, 0, 1)


# speedup vs baseline: 1.3084x; 1.3084x over previous
import functools

import jax
import jax.numpy as jnp
from jax import lax
from jax.experimental import pallas as pl
from jax.experimental.pallas import tpu as pltpu
from jax.experimental.pallas import tpu_sc as plsc

_INFO = plsc.get_sparse_core_info()
_NC = _INFO.num_cores
_NS = _INFO.num_subcores

_V = 1000000
_D = 32
_R = 50
_N = 16384
_NBLK = _V // 128
_TAIL = _V - _NBLK * 128
_TROWS = _TAIL * _D // 128
_SROWS = _V * _D // 128
_NW = _NC * _NS
_NPW = _N // _NW
_C = 128
_NCH = _R * (_NPW // _C)
_SB = _NBLK // 2
_SBQ, _SBR = divmod(_SB, _NS)


def _make_kernel():
    mesh = plsc.VectorSubcoreMesh(core_axis_name="c", subcore_axis_name="s")

    @functools.partial(
        pl.kernel,
        mesh=mesh,
        out_type=jax.ShapeDtypeStruct((_R, _D, _N), jnp.float32),
        scratch_types=[
            pltpu.HBM((_NC, _SROWS, 128), jnp.float32),
            pltpu.VMEM((_D, 256), jnp.float32),
            pltpu.VMEM((_D, 256), jnp.float32),
            pltpu.VMEM((64, 128), jnp.float32),
            pltpu.VMEM((64, 128), jnp.float32),
            pltpu.VMEM((_C,), jnp.int32),
            pltpu.VMEM((_C,), jnp.int32),
            pltpu.VMEM((_C,), jnp.int32),
            pltpu.VMEM((_C,), jnp.int32),
            pltpu.VMEM((_C,), jnp.int32),
            pltpu.VMEM((_C,), jnp.int32),
            pltpu.VMEM((_C, 128), jnp.float32),
            pltpu.VMEM((_C, 128), jnp.float32),
            pltpu.VMEM((_D, _C), jnp.float32),
            pltpu.VMEM((_D, _C), jnp.float32),
            pltpu.SemaphoreType.DMA,
            pltpu.SemaphoreType.DMA,
            pltpu.SemaphoreType.DMA,
            pltpu.SemaphoreType.DMA,
            pltpu.SemaphoreType.DMA,
            pltpu.SemaphoreType.DMA,
            pltpu.SemaphoreType.DMA,
            pltpu.SemaphoreType.DMA,
        ],
        compiler_params=pltpu.CompilerParams(needs_layout_passes=False),
    )
    def k(wt_t, idx_t, tail_rows, out_t, scr, tbuf_0, tbuf_1, obuf1_0,
          obuf1_1, ibuf_0, ibuf_1, qbuf_0, qbuf_1, cbuf_0, cbuf_1, gbuf_0,
          gbuf_1, obuf2_0, obuf2_1, in0, in1, o0, o1, g0, g1, w0, w1):
        tbuf = [tbuf_0, tbuf_1]
        obuf1 = [obuf1_0, obuf1_1]
        ibuf = [ibuf_0, ibuf_1]
        qbuf = [qbuf_0, qbuf_1]
        cbuf = [cbuf_0, cbuf_1]
        gbuf = [gbuf_0, gbuf_1]
        obuf2 = [obuf2_0, obuf2_1]
        c = lax.axis_index("c")
        s = lax.axis_index("s")
        row_lo = lax.iota(jnp.int32, 16)
        p1_in = [in0, in1]
        p1_out = [o0, o1]
        p2_g = [g0, g1]
        p2_w = [w0, w1]

        base = s * _SBQ + jnp.minimum(s, _SBR)
        nb = _SBQ + jnp.where(s < _SBR, 1, 0)

        def p1_start_in(t, p):
            pltpu.async_copy(wt_t.at[:, pl.ds((base + t) * 256, 256)],
                             tbuf[p], p1_in[p])

        def p1_wait_in(p):
            pltpu.make_async_copy(wt_t.at[:, pl.ds(0, 256)], tbuf[p],
                                  p1_in[p]).wait()

        def p1_shuffle(p):
            tb = tbuf[p]
            ob = obuf1[p]

            def sh_body(k_, carry):
                for lg in range(8):
                    l0 = lg * 16
                    rv = row_lo + l0 % 32
                    cv = jnp.full((16,), 4 * k_ + l0 // 32, jnp.int32)
                    ob[k_, pl.ds(l0, 16)] = plsc.load_gather(tb, [rv, cv])
                return carry

            lax.fori_loop(0, 64, sh_body, 0)

        def p1_start_out(t, p):
            pltpu.async_copy(obuf1[p],
                             scr.at[c, pl.ds((base + t) * 64, 64), :],
                             p1_out[p])

        def p1_wait_out(p):
            pltpu.make_async_copy(obuf1[p], scr.at[c, pl.ds(0, 64), :],
                                  p1_out[p]).wait()

        def p1_half(t, p):
            @pl.when(t < nb)
            def _():
                @pl.when(t + 1 < nb)
                def _():
                    p1_start_in(t + 1, 1 - p)

                p1_wait_in(p)

                @pl.when(t >= 2)
                def _():
                    p1_wait_out(p)

                p1_shuffle(p)
                p1_start_out(t, p)

        p1_start_in(0, 0)

        def p1_body(pt, carry):
            p1_half(2 * pt, 0)
            p1_half(2 * pt + 1, 1)
            return carry

        lax.fori_loop(0, (_SBQ + 2) // 2, p1_body, 0)
        p1_wait_out(0)
        p1_wait_out(1)

        @pl.when(s == 0)
        def _tail():
            pltpu.sync_copy(tail_rows, obuf1[0].at[pl.ds(0, _TROWS), :])
            pltpu.sync_copy(obuf1[0].at[pl.ds(0, _TROWS), :],
                            scr.at[c, pl.ds(32 * _NBLK, _TROWS), :])

        plsc.subcore_barrier()

        wid = s * _NC + c
        n0w = wid * _NPW
        scr_c = scr.at[c]

        def p2_start(i, p):
            r = i // (_NPW // _C)
            ch = i % (_NPW // _C)
            n0 = n0w + ch * _C
            ib = ibuf[p]
            pltpu.sync_copy(idx_t.at[r, pl.ds(n0, _C)], ib)
            for m in range(_C // 16):
                v = ib[pl.ds(m * 16, 16)]
                qbuf[p][pl.ds(m * 16, 16)] = lax.shift_right_logical(v, 2)
                cbuf[p][pl.ds(m * 16, 16)] = lax.shift_left(
                    jnp.bitwise_and(v, 3), 5)
            for h in range(_C // 128):
                pltpu.async_copy(
                    scr_c.at[qbuf[p].at[pl.ds(h * 128, 128)]],
                    gbuf[p].at[pl.ds(h * 128, 128), :], p2_g[p])

        def p2_finish(i, p):
            for h in range(_C // 128):
                pltpu.make_async_copy(
                    scr_c.at[pl.ds(0, 128), :],
                    gbuf[p].at[pl.ds(h * 128, 128), :], p2_g[p]).wait()

            @pl.when(i >= 2)
            def _():
                pltpu.make_async_copy(obuf2[p],
                                      out_t.at[0, :, pl.ds(0, _C)],
                                      p2_w[p]).wait()

            gb = gbuf[p]
            ob = obuf2[p]

            def mg_body(mg, carry2):
                m0 = mg * 16
                rv = row_lo + m0
                cb = cbuf[p][pl.ds(m0, 16)]
                for j in range(_D):
                    ob[j, pl.ds(m0, 16)] = plsc.load_gather(gb, [rv, cb + j])
                return carry2

            lax.fori_loop(0, _C // 16, mg_body, 0)
            r = i // (_NPW // _C)
            ch = i % (_NPW // _C)
            n0 = n0w + ch * _C
            pltpu.async_copy(ob, out_t.at[r, :, pl.ds(n0, _C)], p2_w[p])

        p2_start(0, 0)

        def p2_body(pi, carry):
            i0 = 2 * pi
            p2_start(i0 + 1, 1)
            p2_finish(i0, 0)

            @pl.when(i0 + 2 < _NCH)
            def _():
                p2_start(i0 + 2, 0)

            p2_finish(i0 + 1, 1)
            return carry

        lax.fori_loop(0, _NCH // 2, p2_body, 0)
        pltpu.make_async_copy(obuf2[0], out_t.at[0, :, pl.ds(0, _C)],
                              p2_w[0]).wait()
        pltpu.make_async_copy(obuf2[1], out_t.at[0, :, pl.ds(0, _C)],
                              p2_w[1]).wait()

    return k


_GATHER = _make_kernel()


def kernel(weights, indices):
    idx_t = indices.astype(jnp.int32).T
    tail_rows = weights[_NBLK * 128:].reshape(_TROWS, 128)
    out_t = _GATHER(weights.T, idx_t, tail_rows)
    return out_t.transpose(2, 0, 1)


# speedup vs baseline: 2.5777x; 1.9701x over previous
import functools

import jax
import jax.numpy as jnp
from jax import lax
from jax.experimental import pallas as pl
from jax.experimental.pallas import tpu as pltpu
from jax.experimental.pallas import tpu_sc as plsc

_INFO = plsc.get_sparse_core_info()
_NC = _INFO.num_cores
_NS = _INFO.num_subcores

_V = 1000000
_D = 32
_R = 50
_N = 16384
_SROWS = _V * _D // 128
_NW = _NC * _NS
_NPW = _N // _NW
_C = 128
_NCH = _R * (_NPW // _C)


def _make_kernel():
    mesh = plsc.VectorSubcoreMesh(core_axis_name="c", subcore_axis_name="s")

    @functools.partial(
        pl.kernel,
        mesh=mesh,
        out_type=jax.ShapeDtypeStruct((_R, _D, _N), jnp.float32),
        scratch_types=[
            pltpu.VMEM((_R, _NPW), jnp.int32),
            pltpu.VMEM((_C,), jnp.int32),
            pltpu.VMEM((_C,), jnp.int32),
            pltpu.VMEM((_C,), jnp.int32),
            pltpu.VMEM((_C,), jnp.int32),
            pltpu.VMEM((_C, 128), jnp.float32),
            pltpu.VMEM((_C, 128), jnp.float32),
            pltpu.VMEM((_D, _C), jnp.float32),
            pltpu.VMEM((_D, _C), jnp.float32),
            pltpu.SemaphoreType.DMA,
            pltpu.SemaphoreType.DMA,
            pltpu.SemaphoreType.DMA,
            pltpu.SemaphoreType.DMA,
        ],
        compiler_params=pltpu.CompilerParams(needs_layout_passes=False),
    )
    def k(tab, idx_t, out_t, ibig, qbuf_0, qbuf_1, cbuf_0, cbuf_1, gbuf_0,
          gbuf_1, obuf_0, obuf_1, g0, g1, w0, w1):
        qbuf = [qbuf_0, qbuf_1]
        cbuf = [cbuf_0, cbuf_1]
        gbuf = [gbuf_0, gbuf_1]
        obuf = [obuf_0, obuf_1]
        p2_g = [g0, g1]
        p2_w = [w0, w1]
        c = lax.axis_index("c")
        s = lax.axis_index("s")
        row_lo = lax.iota(jnp.int32, 16)
        wid = s * _NC + c
        n0w = wid * _NPW
        pltpu.sync_copy(idx_t.at[:, pl.ds(n0w, _NPW)], ibig)

        def p2_start(i, p):
            r = i // (_NPW // _C)
            ch = i % (_NPW // _C)
            for m in range(_C // 16):
                v = ibig[r, pl.ds(ch * _C + m * 16, 16)]
                qbuf[p][pl.ds(m * 16, 16)] = lax.shift_right_logical(v, 2)
                cbuf[p][pl.ds(m * 16, 16)] = lax.shift_left(
                    jnp.bitwise_and(v, 3), 5)
            pltpu.async_copy(tab.at[qbuf[p]], gbuf[p], p2_g[p])

        def p2_finish(i, p):
            pltpu.make_async_copy(tab.at[pl.ds(0, _C), :], gbuf[p],
                                  p2_g[p]).wait()

            @pl.when(i >= 2)
            def _():
                pltpu.make_async_copy(obuf[p], out_t.at[0, :, pl.ds(0, _C)],
                                      p2_w[p]).wait()

            gb = gbuf[p]
            ob = obuf[p]

            def mg_body(mg, carry2):
                m0 = mg * 16
                rv = row_lo + m0
                cb = cbuf[p][pl.ds(m0, 16)]
                for j in range(_D):
                    ob[j, pl.ds(m0, 16)] = plsc.load_gather(gb, [rv, cb + j])
                return carry2

            lax.fori_loop(0, _C // 16, mg_body, 0)
            r = i // (_NPW // _C)
            ch = i % (_NPW // _C)
            n0 = n0w + ch * _C
            pltpu.async_copy(ob, out_t.at[r, :, pl.ds(n0, _C)], p2_w[p])

        p2_start(0, 0)

        def p2_body(pi, carry):
            i0 = 2 * pi
            p2_start(i0 + 1, 1)
            p2_finish(i0, 0)

            @pl.when(i0 + 2 < _NCH)
            def _():
                p2_start(i0 + 2, 0)

            p2_finish(i0 + 1, 1)
            return carry

        lax.fori_loop(0, _NCH // 2, p2_body, 0)
        pltpu.make_async_copy(obuf[0], out_t.at[0, :, pl.ds(0, _C)],
                              p2_w[0]).wait()
        pltpu.make_async_copy(obuf[1], out_t.at[0, :, pl.ds(0, _C)],
                              p2_w[1]).wait()

    return k


_GATHER = _make_kernel()


def kernel(weights, indices):
    tab = weights.reshape(_SROWS, 128)
    idx_t = indices.astype(jnp.int32).T
    out_t = _GATHER(tab, idx_t)
    return out_t.transpose(2, 0, 1)


# speedup vs baseline: 2.5783x; 1.0003x over previous
import functools

import jax
import jax.numpy as jnp
from jax import lax
from jax.experimental import pallas as pl
from jax.experimental.pallas import tpu as pltpu
from jax.experimental.pallas import tpu_sc as plsc

_INFO = plsc.get_sparse_core_info()
_NC = _INFO.num_cores
_NS = _INFO.num_subcores

_V = 1000000
_D = 32
_R = 50
_N = 16384
_SROWS = _V * _D // 128
_NW = _NC * _NS
_NPW = _N // _NW
_C = 256
_NCH = _R * (_NPW // _C)


def _make_kernel():
    mesh = plsc.VectorSubcoreMesh(core_axis_name="c", subcore_axis_name="s")

    @functools.partial(
        pl.kernel,
        mesh=mesh,
        out_type=jax.ShapeDtypeStruct((_R, _D, _N), jnp.float32),
        scratch_types=[
            pltpu.VMEM((_R, _NPW), jnp.int32),
            pltpu.VMEM((_C,), jnp.int32),
            pltpu.VMEM((_C,), jnp.int32),
            pltpu.VMEM((_C,), jnp.int32),
            pltpu.VMEM((_C,), jnp.int32),
            pltpu.VMEM((_C, 128), jnp.float32),
            pltpu.VMEM((_C, 128), jnp.float32),
            pltpu.VMEM((_D, _C), jnp.float32),
            pltpu.VMEM((_D, _C), jnp.float32),
            pltpu.SemaphoreType.DMA,
            pltpu.SemaphoreType.DMA,
            pltpu.SemaphoreType.DMA,
            pltpu.SemaphoreType.DMA,
        ],
        compiler_params=pltpu.CompilerParams(needs_layout_passes=False),
    )
    def k(tab, idx_t, out_t, ibig, qbuf_0, qbuf_1, cbuf_0, cbuf_1, gbuf_0,
          gbuf_1, obuf_0, obuf_1, g0, g1, w0, w1):
        qbuf = [qbuf_0, qbuf_1]
        cbuf = [cbuf_0, cbuf_1]
        gbuf = [gbuf_0, gbuf_1]
        obuf = [obuf_0, obuf_1]
        p2_g = [g0, g1]
        p2_w = [w0, w1]
        c = lax.axis_index("c")
        s = lax.axis_index("s")
        row_lo = lax.iota(jnp.int32, 16)
        wid = s * _NC + c
        n0w = wid * _NPW
        pltpu.sync_copy(idx_t.at[:, pl.ds(n0w, _NPW)], ibig)

        def p2_start(i, p):
            r = i // (_NPW // _C)
            ch = i % (_NPW // _C)
            for m in range(_C // 16):
                v = ibig[r, pl.ds(ch * _C + m * 16, 16)]
                qbuf[p][pl.ds(m * 16, 16)] = lax.shift_right_logical(v, 2)
                cbuf[p][pl.ds(m * 16, 16)] = lax.shift_left(
                    jnp.bitwise_and(v, 3), 5)
            pltpu.async_copy(tab.at[qbuf[p]], gbuf[p], p2_g[p])

        def p2_finish(i, p):
            pltpu.make_async_copy(tab.at[pl.ds(0, _C), :], gbuf[p],
                                  p2_g[p]).wait()

            @pl.when(i >= 2)
            def _():
                pltpu.make_async_copy(obuf[p], out_t.at[0, :, pl.ds(0, _C)],
                                      p2_w[p]).wait()

            gb = gbuf[p]
            ob = obuf[p]

            def mg_body(mg, carry2):
                m0 = mg * 16
                rv = row_lo + m0
                cb = cbuf[p][pl.ds(m0, 16)]
                for j in range(_D):
                    ob[j, pl.ds(m0, 16)] = plsc.load_gather(gb, [rv, cb + j])
                return carry2

            lax.fori_loop(0, _C // 16, mg_body, 0)
            r = i // (_NPW // _C)
            ch = i % (_NPW // _C)
            n0 = n0w + ch * _C
            pltpu.async_copy(ob, out_t.at[r, :, pl.ds(n0, _C)], p2_w[p])

        p2_start(0, 0)

        def p2_body(pi, carry):
            i0 = 2 * pi
            p2_start(i0 + 1, 1)
            p2_finish(i0, 0)

            @pl.when(i0 + 2 < _NCH)
            def _():
                p2_start(i0 + 2, 0)

            p2_finish(i0 + 1, 1)
            return carry

        lax.fori_loop(0, _NCH // 2, p2_body, 0)
        pltpu.make_async_copy(obuf[0], out_t.at[0, :, pl.ds(0, _C)],
                              p2_w[0]).wait()
        pltpu.make_async_copy(obuf[1], out_t.at[0, :, pl.ds(0, _C)],
                              p2_w[1]).wait()

    return k


_GATHER = _make_kernel()


def kernel(weights, indices):
    tab = weights.reshape(_SROWS, 128)
    idx_t = indices.astype(jnp.int32).T
    out_t = _GATHER(tab, idx_t)
    return out_t.transpose(2, 0, 1)


# speedup vs baseline: 3.9040x; 1.5141x over previous
import functools

import jax
import jax.numpy as jnp
from jax import lax
from jax.experimental import pallas as pl
from jax.experimental.pallas import tpu as pltpu
from jax.experimental.pallas import tpu_sc as plsc

_INFO = plsc.get_sparse_core_info()
_NC = _INFO.num_cores
_NS = _INFO.num_subcores

_V = 1000000
_D = 32
_R = 50
_N = 16384
_SROWS = _V * _D // 128
_NW = _NC * _NS
_NPW = _N // _NW
_C = 256
_NCH = _R * (_NPW // _C)


def _make_kernel():
    mesh = plsc.VectorSubcoreMesh(core_axis_name="c", subcore_axis_name="s")

    @functools.partial(
        pl.kernel,
        mesh=mesh,
        out_type=jax.ShapeDtypeStruct((_R, _D, _N), jnp.float32),
        scratch_types=[
            pltpu.VMEM((_R, _NPW), jnp.int32),
            pltpu.VMEM((_C,), jnp.int32),
            pltpu.VMEM((_C,), jnp.int32),
            pltpu.VMEM((_C,), jnp.int32),
            pltpu.VMEM((_C,), jnp.int32),
            pltpu.VMEM((_C, 128), jnp.float32),
            pltpu.VMEM((_C, 128), jnp.float32),
            pltpu.VMEM((_D, _C), jnp.float32),
            pltpu.VMEM((_D, _C), jnp.float32),
            pltpu.SemaphoreType.DMA,
            pltpu.SemaphoreType.DMA,
            pltpu.SemaphoreType.DMA,
            pltpu.SemaphoreType.DMA,
        ],
        compiler_params=pltpu.CompilerParams(needs_layout_passes=False),
    )
    def k(tab, idx_t, out_t, ibig, qbuf_0, qbuf_1, cbuf_0, cbuf_1, gbuf_0,
          gbuf_1, obuf_0, obuf_1, g0, g1, w0, w1):
        qbuf = [qbuf_0, qbuf_1]
        cbuf = [cbuf_0, cbuf_1]
        gbuf = [gbuf_0, gbuf_1]
        obuf = [obuf_0, obuf_1]
        p2_g = [g0, g1]
        p2_w = [w0, w1]
        c = lax.axis_index("c")
        s = lax.axis_index("s")
        row_lo = lax.iota(jnp.int32, 16)
        joffs = [jnp.bitwise_and(row_lo + d, 15) for d in range(16)]
        wid = s * _NC + c
        n0w = wid * _NPW
        pltpu.sync_copy(idx_t.at[:, pl.ds(n0w, _NPW)], ibig)

        def p2_start(i, p):
            r = i // (_NPW // _C)
            ch = i % (_NPW // _C)
            for m in range(_C // 16):
                v = ibig[r, pl.ds(ch * _C + m * 16, 16)]
                qbuf[p][pl.ds(m * 16, 16)] = lax.shift_right_logical(v, 2)
                cbuf[p][pl.ds(m * 16, 16)] = lax.shift_left(
                    jnp.bitwise_and(v, 3), 5)
            pltpu.async_copy(tab.at[qbuf[p]], gbuf[p], p2_g[p])

        def p2_finish(i, p):
            pltpu.make_async_copy(tab.at[pl.ds(0, _C), :], gbuf[p],
                                  p2_g[p]).wait()

            @pl.when(i >= 2)
            def _():
                pltpu.make_async_copy(obuf[p], out_t.at[0, :, pl.ds(0, _C)],
                                      p2_w[p]).wait()

            gb = gbuf[p]
            ob = obuf[p]

            def mg_body(mg, carry2):
                m0 = mg * 16
                mv = row_lo + m0
                cb = cbuf[p][pl.ds(m0, 16)]
                for j0 in (0, 16):
                    cbj = cb + j0
                    for d in range(16):
                        joff = joffs[d]
                        vals = plsc.load_gather(gb, [mv, cbj + joff])
                        plsc.store_scatter(ob, [joff + j0, mv], vals)
                return carry2

            lax.fori_loop(0, _C // 16, mg_body, 0)
            r = i // (_NPW // _C)
            ch = i % (_NPW // _C)
            n0 = n0w + ch * _C
            pltpu.async_copy(ob, out_t.at[r, :, pl.ds(n0, _C)], p2_w[p])

        p2_start(0, 0)

        def p2_body(pi, carry):
            i0 = 2 * pi
            p2_start(i0 + 1, 1)
            p2_finish(i0, 0)

            @pl.when(i0 + 2 < _NCH)
            def _():
                p2_start(i0 + 2, 0)

            p2_finish(i0 + 1, 1)
            return carry

        lax.fori_loop(0, _NCH // 2, p2_body, 0)
        pltpu.make_async_copy(obuf[0], out_t.at[0, :, pl.ds(0, _C)],
                              p2_w[0]).wait()
        pltpu.make_async_copy(obuf[1], out_t.at[0, :, pl.ds(0, _C)],
                              p2_w[1]).wait()

    return k


_GATHER = _make_kernel()


def kernel(weights, indices):
    tab = weights.reshape(_SROWS, 128)
    idx_t = indices.astype(jnp.int32).T
    out_t = _GATHER(tab, idx_t)
    return out_t.transpose(2, 0, 1)
